# baseline (speedup 1.0000x reference)
import functools
import math

import jax
import jax.numpy as jnp
from jax import lax
from jax.experimental import pallas as pl
from jax.experimental.pallas import tpu as pltpu
from jax.experimental.pallas import tpu_sc as plsc

_VOCAB = 100000
_D = 2048
_BATCH = 4
_SEQ = 2048
_B = _BATCH * _SEQ
_SCALE = math.sqrt(_D)

_NC = 2
_NS = 16
_NW = _NC * _NS
_BPW = _B // _NW
_LANES = 16
_CHUNK = 8
_NCH = _BPW // _CHUNK
_NT = _NCH // 2
_VPR = _D // _LANES
_SROWS = 2 * _CHUNK

_mesh = plsc.VectorSubcoreMesh(core_axis_name="c", subcore_axis_name="s")

_GDN = lax.GatherDimensionNumbers(
    offset_dims=(), collapsed_slice_dims=(0,), start_index_map=(0,)
)


def _splat(vec, lane):
    return lax.gather(
        vec,
        jnp.full((_LANES, 1), lane, jnp.int32),
        _GDN,
        slice_sizes=(1,),
        mode=lax.GatherScatterMode.PROMISE_IN_BOUNDS,
    )


@functools.partial(
    pl.kernel,
    mesh=_mesh,
    out_type=jax.ShapeDtypeStruct((_B * _D,), jnp.float32),
    scratch_types=[
        pltpu.VMEM((_BPW,), jnp.int32),
        pltpu.VMEM((_CHUNK, _D), jnp.float32),
        pltpu.VMEM((_CHUNK, _D), jnp.float32),
        pltpu.VMEM((_SROWS * _D,), jnp.float32),
        pltpu.VMEM((_SROWS * _D,), jnp.float32),
        pltpu.SemaphoreType.DMA,
        pltpu.SemaphoreType.DMA,
        pltpu.SemaphoreType.DMA,
        pltpu.SemaphoreType.DMA,
    ],
)
def _emb_lookup(
    idx_hbm, table_hbm, out_hbm,
    idx_v, g0, g1, s0, s1, gsem0, gsem1, ssem0, ssem1,
):
    wid = lax.axis_index("s") * _NC + lax.axis_index("c")
    base = wid * _BPW
    bi = base // _SEQ
    sbase = base % _SEQ
    pltpu.sync_copy(idx_hbm.at[bi, pl.ds(sbase, _BPW)], idx_v)

    gbuf = (g0, g1)
    sbuf = (s0, s1)
    gsem = (gsem0, gsem1)
    ssem = (ssem0, ssem1)

    def issue_gather(k, b):
        pltpu.async_copy(
            table_hbm.at[idx_v.at[pl.ds(k * _CHUNK, _CHUNK)]], gbuf[b], gsem[b]
        )

    def wait_gather(b):
        pltpu.make_async_copy(
            table_hbm.at[idx_v.at[pl.ds(0, _CHUNK)]], gbuf[b], gsem[b]
        ).wait()

    def issue_store(t, sb):
        pltpu.async_copy(
            sbuf[sb],
            out_hbm.at[pl.ds((base + t * _SROWS) * _D, _SROWS * _D)],
            ssem[sb],
        )

    def wait_store(sb):
        pltpu.make_async_copy(
            sbuf[sb], out_hbm.at[pl.ds(0, _SROWS * _D)], ssem[sb]
        ).wait()

    def process(t, sb, first_s=False, last_g=False):
        for b in range(2):
            k = 2 * t + b
            wait_gather(b)
            if b == 0 and not first_s:
                wait_store(sb)
            iv = idx_v[pl.ds(t * _LANES, _LANES)]
            sv = jnp.where(iv != 0, jnp.float32(_SCALE), jnp.float32(0.0))
            splats = [_splat(sv, b * _CHUNK + rr) for rr in range(_CHUNK)]

            def jbody(j, c):
                off = j * _LANES
                for rr in range(_CHUNK):
                    sbuf[sb][pl.ds((b * _CHUNK + rr) * _D + off, _LANES)] = (
                        gbuf[b][rr, pl.ds(off, _LANES)] * splats[rr]
                    )
                return c

            lax.fori_loop(0, _VPR, jbody, 0)
            if not last_g:
                issue_gather(k + 2, b)
        issue_store(t, sb)

    issue_gather(0, 0)
    issue_gather(1, 1)
    process(0, 0, first_s=True)
    process(1, 1, first_s=True)

    def tbody(u, c):
        process(2 * u, 0)
        process(2 * u + 1, 1)
        return c

    lax.fori_loop(1, _NT // 2 - 1, tbody, 0)

    process(_NT - 2, 0)
    process(_NT - 1, 1, last_g=True)
    wait_store(0)
    wait_store(1)


def kernel(x, embedding_weights):
    out = _emb_lookup(x.astype(jnp.int32), embedding_weights)
    return out.reshape(_BATCH, _SEQ, 1, _D)


# speedup vs baseline: 1.3903x; 1.3903x over previous
import functools
import math

import jax
import jax.numpy as jnp
from jax import lax
from jax.experimental import pallas as pl
from jax.experimental.pallas import tpu as pltpu
from jax.experimental.pallas import tpu_sc as plsc

_VOCAB = 100000
_D = 2048
_BATCH = 4
_SEQ = 2048
_B = _BATCH * _SEQ
_SCALE = math.sqrt(_D)

_NC = 2
_NS = 16
_NW = _NC * _NS
_BPW = _B // _NW
_LANES = 16
_CHUNK = 8
_NCH = _BPW // _CHUNK
_NT = _NCH // 2
_VPR = _D // _LANES

_mesh = plsc.VectorSubcoreMesh(core_axis_name="c", subcore_axis_name="s")

_GDN = lax.GatherDimensionNumbers(
    offset_dims=(), collapsed_slice_dims=(0,), start_index_map=(0,)
)


def _splat(vec, lane):
    return lax.gather(
        vec,
        jnp.full((_LANES, 1), lane, jnp.int32),
        _GDN,
        slice_sizes=(1,),
        mode=lax.GatherScatterMode.PROMISE_IN_BOUNDS,
    )


@functools.partial(
    pl.kernel,
    mesh=_mesh,
    out_type=jax.ShapeDtypeStruct((_BATCH, _SEQ, _D), jnp.float32),
    scratch_types=[
        pltpu.VMEM((_BPW,), jnp.int32),
        pltpu.VMEM((_CHUNK, _D), jnp.float32),
        pltpu.VMEM((_CHUNK, _D), jnp.float32),
        pltpu.VMEM((_CHUNK, _D), jnp.float32),
        pltpu.VMEM((_CHUNK, _D), jnp.float32),
        pltpu.SemaphoreType.DMA,
        pltpu.SemaphoreType.DMA,
        pltpu.SemaphoreType.DMA,
        pltpu.SemaphoreType.DMA,
    ],
)
def _emb_lookup(
    idx_hbm, table_hbm, out_hbm,
    idx_v, g0, g1, s0, s1, gsem0, gsem1, ssem0, ssem1,
):
    wid = lax.axis_index("s") * _NC + lax.axis_index("c")
    base = wid * _BPW
    bi = base // _SEQ
    sbase = base % _SEQ
    pltpu.sync_copy(idx_hbm.at[bi, pl.ds(sbase, _BPW)], idx_v)

    gbuf = (g0, g1)
    sbuf = (s0, s1)
    gsem = (gsem0, gsem1)
    ssem = (ssem0, ssem1)

    def issue_gather(k, b):
        pltpu.async_copy(
            table_hbm.at[idx_v.at[pl.ds(k * _CHUNK, _CHUNK)]], gbuf[b], gsem[b]
        )

    def wait_gather(b):
        pltpu.make_async_copy(
            table_hbm.at[idx_v.at[pl.ds(0, _CHUNK)]], gbuf[b], gsem[b]
        ).wait()

    def issue_store(k, b):
        pltpu.async_copy(
            sbuf[b], out_hbm.at[bi, pl.ds(sbase + k * _CHUNK, _CHUNK)], ssem[b]
        )

    def wait_store(b):
        pltpu.make_async_copy(
            sbuf[b], out_hbm.at[bi, pl.ds(sbase, _CHUNK)], ssem[b]
        ).wait()

    def process(t, b, first=False, last=False):
        k = 2 * t + b
        wait_gather(b)
        if not first:
            wait_store(b)
        iv = idx_v[pl.ds(t * _LANES, _LANES)]
        sv = jnp.where(iv != 0, jnp.float32(_SCALE), jnp.float32(0.0))
        splats = [_splat(sv, b * _CHUNK + rr) for rr in range(_CHUNK)]

        def jbody(j, c):
            sl = pl.ds(j * _LANES, _LANES)
            for rr in range(_CHUNK):
                sbuf[b][rr, sl] = gbuf[b][rr, sl] * splats[rr]
            return c

        lax.fori_loop(0, _VPR, jbody, 0)
        if not last:
            issue_gather(k + 2, b)
        issue_store(k, b)

    issue_gather(0, 0)
    issue_gather(1, 1)
    process(0, 0, first=True)
    process(0, 1, first=True)

    def tbody(t, c):
        process(t, 0)
        process(t, 1)
        return c

    lax.fori_loop(1, _NT - 1, tbody, 0)

    process(_NT - 1, 0, last=True)
    process(_NT - 1, 1, last=True)
    wait_store(0)
    wait_store(1)


def kernel(x, embedding_weights):
    out = _emb_lookup(x.astype(jnp.int32), embedding_weights)
    return jnp.expand_dims(out, 2)


# speedup vs baseline: 1.4013x; 1.0079x over previous
import functools
import math

import jax
import jax.numpy as jnp
from jax import lax
from jax.experimental import pallas as pl
from jax.experimental.pallas import tpu as pltpu
from jax.experimental.pallas import tpu_sc as plsc

_VOCAB = 100000
_D = 2048
_BATCH = 4
_SEQ = 2048
_B = _BATCH * _SEQ
_SCALE = math.sqrt(_D)

_NC = 2
_NS = 16
_NW = _NC * _NS
_BPW = _B // _NW
_LANES = 16
_CHUNK = 8
_NCH = _BPW // _CHUNK
_NT = _NCH // 2
_VPR = _D // _LANES

_mesh = plsc.VectorSubcoreMesh(core_axis_name="c", subcore_axis_name="s")

_GDN = lax.GatherDimensionNumbers(
    offset_dims=(), collapsed_slice_dims=(0,), start_index_map=(0,)
)


def _splat(vec, lane):
    return lax.gather(
        vec,
        jnp.full((_LANES, 1), lane, jnp.int32),
        _GDN,
        slice_sizes=(1,),
        mode=lax.GatherScatterMode.PROMISE_IN_BOUNDS,
    )


@functools.partial(
    pl.kernel,
    mesh=_mesh,
    out_type=jax.ShapeDtypeStruct((_BATCH, _SEQ, _D), jnp.float32),
    scratch_types=[
        pltpu.VMEM((_BPW,), jnp.int32),
        pltpu.VMEM((_CHUNK, _D), jnp.float32),
        pltpu.VMEM((_CHUNK, _D), jnp.float32),
        pltpu.VMEM((_CHUNK, _D), jnp.float32),
        pltpu.VMEM((_CHUNK, _D), jnp.float32),
        pltpu.VMEM((_CHUNK, _D), jnp.float32),
        pltpu.SemaphoreType.DMA,
        pltpu.SemaphoreType.DMA,
        pltpu.SemaphoreType.DMA,
        pltpu.SemaphoreType.DMA,
        pltpu.SemaphoreType.DMA,
    ],
)
def _emb_lookup(
    idx_hbm, table_hbm, out_hbm,
    idx_v, g0, g1, g2, s0, s1, gsem0, gsem1, gsem2, ssem0, ssem1,
):
    wid = lax.axis_index("s") * _NC + lax.axis_index("c")
    base = wid * _BPW
    bi = base // _SEQ
    sbase = base % _SEQ
    pltpu.sync_copy(idx_hbm.at[bi, pl.ds(sbase, _BPW)], idx_v)

    gbuf = (g0, g1, g2)
    sbuf = (s0, s1)
    gsem = (gsem0, gsem1, gsem2)
    ssem = (ssem0, ssem1)

    def issue_gather(k, gb):
        pltpu.async_copy(
            table_hbm.at[idx_v.at[pl.ds(k * _CHUNK, _CHUNK)]],
            gbuf[gb],
            gsem[gb],
        )

    def wait_gather(gb):
        pltpu.make_async_copy(
            table_hbm.at[idx_v.at[pl.ds(0, _CHUNK)]], gbuf[gb], gsem[gb]
        ).wait()

    def issue_store(k, b):
        pltpu.async_copy(
            sbuf[b], out_hbm.at[bi, pl.ds(sbase + k * _CHUNK, _CHUNK)], ssem[b]
        )

    def wait_store(b):
        pltpu.make_async_copy(
            sbuf[b], out_hbm.at[bi, pl.ds(sbase, _CHUNK)], ssem[b]
        ).wait()

    def process(k, gb, sb, b, first=False, last=False):
        t = (k - b) // 2
        wait_gather(gb)
        if not first:
            wait_store(sb)
        iv = idx_v[pl.ds(t * _LANES, _LANES)]
        sv = jnp.where(iv != 0, jnp.float32(_SCALE), jnp.float32(0.0))
        splats = [_splat(sv, b * _CHUNK + rr) for rr in range(_CHUNK)]

        def jbody(j, c):
            sl = pl.ds(j * _LANES, _LANES)
            for rr in range(_CHUNK):
                sbuf[sb][rr, sl] = gbuf[gb][rr, sl] * splats[rr]
            return c

        lax.fori_loop(0, _VPR, jbody, 0)
        if not last:
            issue_gather(k + 3, gb)
        issue_store(k, sb)

    issue_gather(0, 0)
    issue_gather(1, 1)
    issue_gather(2, 2)
    process(0, 0, 0, 0, first=True)
    process(1, 1, 1, 1, first=True)
    process(2, 2, 0, 0)
    process(3, 0, 1, 1)

    def wbody(w, c):
        k = 6 * w + 4
        for i in range(6):
            process(k + i, (4 + i) % 3, i % 2, i % 2)
        return c

    lax.fori_loop(0, (_NCH - 8) // 6, wbody, 0)

    process(_NCH - 4, 1, 0, 0)
    process(_NCH - 3, 2, 1, 1, last=True)
    process(_NCH - 2, 0, 0, 0, last=True)
    process(_NCH - 1, 1, 1, 1, last=True)
    wait_store(0)
    wait_store(1)


def kernel(x, embedding_weights):
    out = _emb_lookup(x.astype(jnp.int32), embedding_weights)
    return jnp.expand_dims(out, 2)
